# Initial kernel scaffold; baseline (speedup 1.0000x reference)
import jax
import jax.numpy as jnp
from jax.experimental import pallas as pl


def kernel(x, edge_index, edge_attr, W1, b1, We, be, Wd, bd, W2, b2, Wc, bc):
    raise NotImplementedError("write your pallas kernel here")


# speedup vs baseline: 11.5452x; 11.5452x over previous
import functools

import jax
import jax.numpy as jnp
from jax import lax
from jax.experimental import pallas as pl
from jax.experimental.pallas import tpu as pltpu
from jax.experimental.pallas import tpu_sc as plsc

_N = 10000
_E = 320000
_DIN = 128
_DH = 64
_DL = 32
_DOUT = 3

_NC, _NS = 2, 16
_NW = _NC * _NS
_CH = 128
_CPW = 80
_EPW = _CH * _CPW
_EP = _NW * _EPW
_SPAN = 640
_NPAD = _NS * _SPAN

_mesh = plsc.VectorSubcoreMesh(core_axis_name="c", subcore_axis_name="s")


def _deg_body(dst_hbm, degp_hbm, idx_v, ones_v, zb, shared):
    c = lax.axis_index("c")
    s = lax.axis_index("s")
    wid = c * _NS + s
    for j in range(8):
        ones_v[0, pl.ds(j * 16, 16)] = jnp.full((16,), 1.0, jnp.float32)
    for j in range(4):
        zb[pl.ds(j * 16, 16)] = jnp.zeros((16,), jnp.float32)
    for j in range(10):
        pltpu.sync_copy(zb, shared.at[pl.ds(s * _SPAN + j * 64, 64)])
    plsc.subcore_barrier()
    base = wid * _EPW

    def chunk(i, carry):
        pltpu.sync_copy(dst_hbm.at[pl.ds(base + i * _CH, _CH)], idx_v.at[0])
        pltpu.sync_copy(ones_v.at[0], shared.at[idx_v.at[0]], add=True)
        return carry

    lax.fori_loop(0, _CPW, chunk, 0)
    plsc.subcore_barrier()
    pltpu.sync_copy(shared.at[pl.ds(s * _SPAN, _SPAN)],
                    degp_hbm.at[c, pl.ds(s * _SPAN, _SPAN)])


_deg_call = pl.kernel(
    _deg_body,
    out_type=jax.ShapeDtypeStruct((_NC, _NPAD), jnp.float32),
    mesh=_mesh,
    scratch_types=[
        pltpu.VMEM((1, _CH), jnp.int32),
        pltpu.VMEM((1, _CH), jnp.float32),
        pltpu.VMEM((64,), jnp.float32),
        pltpu.VMEM_SHARED((_NPAD,), jnp.float32),
    ],
)


def _spmm_body(hp_hbm, src_hbm, dst_hbm, part_hbm, isrc, idst, rows, zb,
               shared, sem):
    c = lax.axis_index("c")
    s = lax.axis_index("s")
    wid = c * _NS + s
    for i in range(16):
        for j in range(4):
            zb[i, pl.ds(j * 16, 16)] = jnp.zeros((16,), jnp.float32)
    for j in range(40):
        pltpu.sync_copy(zb, shared.at[pl.ds(s * _SPAN + j * 16, 16), :])
    plsc.subcore_barrier()
    base = wid * _EPW

    def chunk(i, carry):
        off = base + i * _CH
        pltpu.sync_copy(src_hbm.at[pl.ds(off, _CH)], isrc.at[0])
        pltpu.sync_copy(dst_hbm.at[pl.ds(off, _CH)], idst.at[0])
        pltpu.async_copy(hp_hbm.at[isrc.at[0]], rows, sem).wait()
        pltpu.sync_copy(rows, shared.at[idst.at[0]], add=True)
        return carry

    lax.fori_loop(0, _CPW, chunk, 0)
    plsc.subcore_barrier()
    pltpu.sync_copy(shared.at[pl.ds(s * _SPAN, _SPAN), :],
                    part_hbm.at[c, pl.ds(s * _SPAN, _SPAN), :])


_spmm_call = pl.kernel(
    _spmm_body,
    out_type=jax.ShapeDtypeStruct((_NC, _NPAD, _DH), jnp.float32),
    mesh=_mesh,
    compiler_params=pltpu.CompilerParams(use_tc_tiling_on_sc=False),
    scratch_types=[
        pltpu.VMEM((1, _CH), jnp.int32),
        pltpu.VMEM((1, _CH), jnp.int32),
        pltpu.VMEM((_CH, _DH), jnp.float32),
        pltpu.VMEM((16, _DH), jnp.float32),
        pltpu.VMEM_SHARED((_NPAD, _DH), jnp.float32),
        pltpu.SemaphoreType.DMA,
    ],
)


_BN = 2000


def _dinv_of(degt_blk):
    return lax.rsqrt(degt_blk[:, 0] + degt_blk[:, 1] + 1.0)


def _tc1_body(x_ref, w1_ref, degp_ref, hp_ref):
    dinv = _dinv_of(degp_ref[...])
    h = jnp.dot(x_ref[...], w1_ref[...], preferred_element_type=jnp.float32)
    hp_ref[...] = h * dinv[:, None]


def _tc1(x, W1, degp):
    return pl.pallas_call(
        _tc1_body,
        grid=(_N // _BN,),
        in_specs=[
            pl.BlockSpec((_BN, _DIN), lambda i: (i, 0)),
            pl.BlockSpec((_DIN, _DH), lambda i: (0, 0)),
            pl.BlockSpec((_BN, _NC), lambda i: (i, 0)),
        ],
        out_specs=pl.BlockSpec((_BN, _DH), lambda i: (i, 0)),
        out_shape=jax.ShapeDtypeStruct((_N, _DH), jnp.float32),
    )(x, W1, degp)


def _tc2_body(q_ref, hp_ref, degp_ref, b1_ref, we_ref, be_ref, wd_ref,
              bd_ref, wc_ref, bc_ref, z_ref, pred_ref, hp2_ref):
    dinv = _dinv_of(degp_ref[...])
    s1 = dinv[:, None] * (q_ref[0] + q_ref[1] + hp_ref[...]) + b1_ref[...]
    h1 = jnp.maximum(s1, 0.0)
    z = jnp.dot(h1, we_ref[...], preferred_element_type=jnp.float32) + be_ref[...]
    h2 = jnp.maximum(
        jnp.dot(z, wd_ref[...], preferred_element_type=jnp.float32) + bd_ref[...],
        0.0)
    z_ref[...] = z
    pred_ref[...] = jnp.dot(z, wc_ref[...], preferred_element_type=jnp.float32) + bc_ref[...]
    hp2_ref[...] = h2 * dinv[:, None]


def _tc2(q, hp, degp, b1, We, be, Wd, bd, Wc, bc):
    return pl.pallas_call(
        _tc2_body,
        grid=(_N // _BN,),
        in_specs=[
            pl.BlockSpec((_NC, _BN, _DH), lambda i: (0, i, 0)),
            pl.BlockSpec((_BN, _DH), lambda i: (i, 0)),
            pl.BlockSpec((_BN, _NC), lambda i: (i, 0)),
            pl.BlockSpec((1, _DH), lambda i: (0, 0)),
            pl.BlockSpec((_DH, _DL), lambda i: (0, 0)),
            pl.BlockSpec((1, _DL), lambda i: (0, 0)),
            pl.BlockSpec((_DL, _DH), lambda i: (0, 0)),
            pl.BlockSpec((1, _DH), lambda i: (0, 0)),
            pl.BlockSpec((_DL, _DOUT), lambda i: (0, 0)),
            pl.BlockSpec((1, _DOUT), lambda i: (0, 0)),
        ],
        out_specs=[
            pl.BlockSpec((_BN, _DL), lambda i: (i, 0)),
            pl.BlockSpec((_BN, _DOUT), lambda i: (i, 0)),
            pl.BlockSpec((_BN, _DH), lambda i: (i, 0)),
        ],
        out_shape=[
            jax.ShapeDtypeStruct((_N, _DL), jnp.float32),
            jax.ShapeDtypeStruct((_N, _DOUT), jnp.float32),
            jax.ShapeDtypeStruct((_N, _DH), jnp.float32),
        ],
    )(q, hp, degp, b1, We, be, Wd, bd, Wc, bc)


def _tc3_body(r_ref, hp2_ref, degp_ref, w2_ref, b2_ref, out_ref):
    dinv = _dinv_of(degp_ref[...])
    t = dinv[:, None] * (r_ref[0] + r_ref[1] + hp2_ref[...])
    out_ref[...] = jnp.dot(t, w2_ref[...], preferred_element_type=jnp.float32) + b2_ref[...]


def _tc3(r, hp2, degp, W2, b2):
    return pl.pallas_call(
        _tc3_body,
        grid=(_N // _BN,),
        in_specs=[
            pl.BlockSpec((_NC, _BN, _DH), lambda i: (0, i, 0)),
            pl.BlockSpec((_BN, _DH), lambda i: (i, 0)),
            pl.BlockSpec((_BN, _NC), lambda i: (i, 0)),
            pl.BlockSpec((_DH, _DIN), lambda i: (0, 0)),
            pl.BlockSpec((1, _DIN), lambda i: (0, 0)),
        ],
        out_specs=pl.BlockSpec((_BN, _DIN), lambda i: (i, 0)),
        out_shape=jax.ShapeDtypeStruct((_N, _DIN), jnp.float32),
    )(r, hp2, degp, W2, b2)


def kernel(x, edge_index, edge_attr, W1, b1, We, be, Wd, bd, W2, b2, Wc, bc):
    npad = _EP - _E
    srcp = jnp.concatenate([edge_index[0], jnp.zeros((npad,), jnp.int32)])
    dstp = jnp.concatenate(
        [edge_index[1], jnp.full((npad,), _NPAD - 1, jnp.int32)])

    degp = _deg_call(dstp)
    degt = degp.T
    hp = _tc1(x, W1, degt)
    q = _spmm_call(hp, srcp, dstp)
    z, pred, hp2 = _tc2(q, hp, degt, b1.reshape(1, -1), We,
                        be.reshape(1, -1), Wd, bd.reshape(1, -1), Wc,
                        bc.reshape(1, -1))
    r = _spmm_call(hp2, srcp, dstp)
    x_recon = _tc3(r, hp2, degt, W2, b2.reshape(1, -1))
    return (x_recon, z, pred)


# speedup vs baseline: 17.7710x; 1.5393x over previous
import functools

import jax
import jax.numpy as jnp
from jax import lax
from jax.experimental import pallas as pl
from jax.experimental.pallas import tpu as pltpu
from jax.experimental.pallas import tpu_sc as plsc

_N = 10000
_E = 320000
_DIN = 128
_DH = 64
_DL = 32
_DOUT = 3

_NC, _NS = 2, 16
_NW = _NC * _NS
_CH = 128
_CPW = 80
_EPW = _CH * _CPW
_EP = _NW * _EPW
_SPAN = 640
_NPAD = _NS * _SPAN

_mesh = plsc.VectorSubcoreMesh(core_axis_name="c", subcore_axis_name="s")


def _deg_body(dst_hbm, degp_hbm, idx_v, ones_v, zb, shared):
    c = lax.axis_index("c")
    s = lax.axis_index("s")
    wid = c * _NS + s
    for j in range(8):
        ones_v[0, pl.ds(j * 16, 16)] = jnp.full((16,), 1.0, jnp.float32)
    for j in range(4):
        zb[pl.ds(j * 16, 16)] = jnp.zeros((16,), jnp.float32)
    for j in range(10):
        pltpu.sync_copy(zb, shared.at[pl.ds(s * _SPAN + j * 64, 64)])
    pltpu.sync_copy(dst_hbm.at[pl.ds(wid * _CPW, _CPW)], idx_v)
    plsc.subcore_barrier()

    def chunk(i, carry):
        pltpu.sync_copy(ones_v.at[0], shared.at[idx_v.at[i]], add=True)
        return carry

    lax.fori_loop(0, _CPW, chunk, 0)
    plsc.subcore_barrier()
    pltpu.sync_copy(shared.at[pl.ds(s * _SPAN, _SPAN)],
                    degp_hbm.at[c, pl.ds(s * _SPAN, _SPAN)])


_deg_call = pl.kernel(
    _deg_body,
    out_type=jax.ShapeDtypeStruct((_NC, _NPAD), jnp.float32),
    mesh=_mesh,
    scratch_types=[
        pltpu.VMEM((_CPW, _CH), jnp.int32),
        pltpu.VMEM((1, _CH), jnp.float32),
        pltpu.VMEM((64,), jnp.float32),
        pltpu.VMEM_SHARED((_NPAD,), jnp.float32),
    ],
)


_NBUF = 4


def _spmm_body(hp_hbm, src_hbm, dst_hbm, part_hbm, isrc, idst, rows, zb,
               shared, sem0, sem1, sem2, sem3):
    sems = (sem0, sem1, sem2, sem3)
    c = lax.axis_index("c")
    s = lax.axis_index("s")
    wid = c * _NS + s
    for i in range(16):
        for j in range(4):
            zb[i, pl.ds(j * 16, 16)] = jnp.zeros((16,), jnp.float32)
    for j in range(40):
        pltpu.sync_copy(zb, shared.at[pl.ds(s * _SPAN + j * 16, 16), :])
    pltpu.sync_copy(src_hbm.at[pl.ds(wid * _CPW, _CPW)], isrc)
    pltpu.sync_copy(dst_hbm.at[pl.ds(wid * _CPW, _CPW)], idst)
    plsc.subcore_barrier()

    for b in range(_NBUF):
        pltpu.async_copy(hp_hbm.at[isrc.at[b]], rows.at[b], sems[b])

    def step(k, carry):
        for b in range(_NBUF):
            ch = k * _NBUF + b
            pltpu.make_async_copy(hp_hbm.at[isrc.at[b]], rows.at[b],
                                  sems[b]).wait()
            pltpu.sync_copy(rows.at[b], shared.at[idst.at[ch]], add=True)

            @pl.when(k < _CPW // _NBUF - 1)
            def _():
                pltpu.async_copy(hp_hbm.at[isrc.at[ch + _NBUF]], rows.at[b],
                                 sems[b])
        return carry

    lax.fori_loop(0, _CPW // _NBUF, step, 0)
    plsc.subcore_barrier()
    pltpu.sync_copy(shared.at[pl.ds(s * _SPAN, _SPAN), :],
                    part_hbm.at[c, pl.ds(s * _SPAN, _SPAN), :])


_spmm_call = pl.kernel(
    _spmm_body,
    out_type=jax.ShapeDtypeStruct((_NC, _NPAD, _DH), jnp.float32),
    mesh=_mesh,
    compiler_params=pltpu.CompilerParams(use_tc_tiling_on_sc=False),
    scratch_types=[
        pltpu.VMEM((_CPW, _CH), jnp.int32),
        pltpu.VMEM((_CPW, _CH), jnp.int32),
        pltpu.VMEM((_NBUF, _CH, _DH), jnp.float32),
        pltpu.VMEM((16, _DH), jnp.float32),
        pltpu.VMEM_SHARED((_NPAD, _DH), jnp.float32),
        pltpu.SemaphoreType.DMA,
        pltpu.SemaphoreType.DMA,
        pltpu.SemaphoreType.DMA,
        pltpu.SemaphoreType.DMA,
    ],
)


_BN = 2000


def _dinv_of(degt_blk):
    return lax.rsqrt(degt_blk[:, 0] + degt_blk[:, 1] + 1.0)


def _tc1_body(x_ref, w1_ref, degp_ref, hp_ref):
    dinv = _dinv_of(degp_ref[...])
    h = jnp.dot(x_ref[...], w1_ref[...], preferred_element_type=jnp.float32)
    hp_ref[...] = h * dinv[:, None]


def _tc1(x, W1, degp):
    return pl.pallas_call(
        _tc1_body,
        grid=(_N // _BN,),
        in_specs=[
            pl.BlockSpec((_BN, _DIN), lambda i: (i, 0)),
            pl.BlockSpec((_DIN, _DH), lambda i: (0, 0)),
            pl.BlockSpec((_BN, _NC), lambda i: (i, 0)),
        ],
        out_specs=pl.BlockSpec((_BN, _DH), lambda i: (i, 0)),
        out_shape=jax.ShapeDtypeStruct((_N, _DH), jnp.float32),
    )(x, W1, degp)


def _tc2_body(q_ref, hp_ref, degp_ref, b1_ref, we_ref, be_ref, wd_ref,
              bd_ref, wc_ref, bc_ref, z_ref, pred_ref, hp2_ref):
    dinv = _dinv_of(degp_ref[...])
    s1 = dinv[:, None] * (q_ref[0] + q_ref[1] + hp_ref[...]) + b1_ref[...]
    h1 = jnp.maximum(s1, 0.0)
    z = jnp.dot(h1, we_ref[...], preferred_element_type=jnp.float32) + be_ref[...]
    h2 = jnp.maximum(
        jnp.dot(z, wd_ref[...], preferred_element_type=jnp.float32) + bd_ref[...],
        0.0)
    z_ref[...] = z
    pred_ref[...] = jnp.dot(z, wc_ref[...], preferred_element_type=jnp.float32) + bc_ref[...]
    hp2_ref[...] = h2 * dinv[:, None]


def _tc2(q, hp, degp, b1, We, be, Wd, bd, Wc, bc):
    return pl.pallas_call(
        _tc2_body,
        grid=(_N // _BN,),
        in_specs=[
            pl.BlockSpec((_NC, _BN, _DH), lambda i: (0, i, 0)),
            pl.BlockSpec((_BN, _DH), lambda i: (i, 0)),
            pl.BlockSpec((_BN, _NC), lambda i: (i, 0)),
            pl.BlockSpec((1, _DH), lambda i: (0, 0)),
            pl.BlockSpec((_DH, _DL), lambda i: (0, 0)),
            pl.BlockSpec((1, _DL), lambda i: (0, 0)),
            pl.BlockSpec((_DL, _DH), lambda i: (0, 0)),
            pl.BlockSpec((1, _DH), lambda i: (0, 0)),
            pl.BlockSpec((_DL, _DOUT), lambda i: (0, 0)),
            pl.BlockSpec((1, _DOUT), lambda i: (0, 0)),
        ],
        out_specs=[
            pl.BlockSpec((_BN, _DL), lambda i: (i, 0)),
            pl.BlockSpec((_BN, _DOUT), lambda i: (i, 0)),
            pl.BlockSpec((_BN, _DH), lambda i: (i, 0)),
        ],
        out_shape=[
            jax.ShapeDtypeStruct((_N, _DL), jnp.float32),
            jax.ShapeDtypeStruct((_N, _DOUT), jnp.float32),
            jax.ShapeDtypeStruct((_N, _DH), jnp.float32),
        ],
    )(q, hp, degp, b1, We, be, Wd, bd, Wc, bc)


def _tc3_body(r_ref, hp2_ref, degp_ref, w2_ref, b2_ref, out_ref):
    dinv = _dinv_of(degp_ref[...])
    t = dinv[:, None] * (r_ref[0] + r_ref[1] + hp2_ref[...])
    out_ref[...] = jnp.dot(t, w2_ref[...], preferred_element_type=jnp.float32) + b2_ref[...]


def _tc3(r, hp2, degp, W2, b2):
    return pl.pallas_call(
        _tc3_body,
        grid=(_N // _BN,),
        in_specs=[
            pl.BlockSpec((_NC, _BN, _DH), lambda i: (0, i, 0)),
            pl.BlockSpec((_BN, _DH), lambda i: (i, 0)),
            pl.BlockSpec((_BN, _NC), lambda i: (i, 0)),
            pl.BlockSpec((_DH, _DIN), lambda i: (0, 0)),
            pl.BlockSpec((1, _DIN), lambda i: (0, 0)),
        ],
        out_specs=pl.BlockSpec((_BN, _DIN), lambda i: (i, 0)),
        out_shape=jax.ShapeDtypeStruct((_N, _DIN), jnp.float32),
    )(r, hp2, degp, W2, b2)


def kernel(x, edge_index, edge_attr, W1, b1, We, be, Wd, bd, W2, b2, Wc, bc):
    npad = _EP - _E
    srcp = jnp.concatenate(
        [edge_index[0], jnp.zeros((npad,), jnp.int32)]).reshape(
            _NW * _CPW, _CH)
    dstp = jnp.concatenate(
        [edge_index[1], jnp.full((npad,), _NPAD - 1, jnp.int32)]).reshape(
            _NW * _CPW, _CH)

    degp = _deg_call(dstp)
    degt = degp.T
    hp = _tc1(x, W1, degt)
    q = _spmm_call(hp, srcp, dstp)
    z, pred, hp2 = _tc2(q, hp, degt, b1.reshape(1, -1), We,
                        be.reshape(1, -1), Wd, bd.reshape(1, -1), Wc,
                        bc.reshape(1, -1))
    r = _spmm_call(hp2, srcp, dstp)
    x_recon = _tc3(r, hp2, degt, W2, b2.reshape(1, -1))
    return (x_recon, z, pred)


# speedup vs baseline: 45.8902x; 2.5823x over previous
import functools

import jax
import jax.numpy as jnp
from jax import lax
from jax.experimental import pallas as pl
from jax.experimental.pallas import tpu as pltpu
from jax.experimental.pallas import tpu_sc as plsc

_N = 10000
_E = 320000
_DIN = 128
_DH = 64
_DL = 32
_DOUT = 3

_NC, _NS = 2, 16
_NW = _NC * _NS
_CH = 128
_CPW = 80
_EPW = _CH * _CPW
_EP = _NW * _EPW
_SPAN = 640
_NPAD = _NS * _SPAN

_mesh = plsc.VectorSubcoreMesh(core_axis_name="c", subcore_axis_name="s")


def _deg_body(dst_hbm, degp_hbm, idx_v, ones_v, zb, shared):
    c = lax.axis_index("c")
    s = lax.axis_index("s")
    wid = c * _NS + s
    for j in range(8):
        ones_v[0, pl.ds(j * 16, 16)] = jnp.full((16,), 1.0, jnp.float32)
    for j in range(4):
        zb[pl.ds(j * 16, 16)] = jnp.zeros((16,), jnp.float32)
    for j in range(10):
        pltpu.sync_copy(zb, shared.at[pl.ds(s * _SPAN + j * 64, 64)])
    pltpu.sync_copy(dst_hbm.at[pl.ds(wid * _CPW, _CPW)], idx_v)
    plsc.subcore_barrier()

    def chunk(i, carry):
        pltpu.sync_copy(ones_v.at[0], shared.at[idx_v.at[i]], add=True)
        return carry

    lax.fori_loop(0, _CPW, chunk, 0)
    plsc.subcore_barrier()
    pltpu.sync_copy(shared.at[pl.ds(s * _SPAN, _SPAN)],
                    degp_hbm.at[c, pl.ds(s * _SPAN, _SPAN)])


_deg_call = pl.kernel(
    _deg_body,
    out_type=jax.ShapeDtypeStruct((_NC, _NPAD), jnp.float32),
    mesh=_mesh,
    scratch_types=[
        pltpu.VMEM((_CPW, _CH), jnp.int32),
        pltpu.VMEM((1, _CH), jnp.float32),
        pltpu.VMEM((64,), jnp.float32),
        pltpu.VMEM_SHARED((_NPAD,), jnp.float32),
    ],
)


_NBUF = 4


def _spmm_body(hp_hbm, src_hbm, dst_hbm, part_hbm, isrc, idst, rows, zb,
               shared, sem0, sem1, sem2, sem3):
    sems = (sem0, sem1, sem2, sem3)
    c = lax.axis_index("c")
    s = lax.axis_index("s")
    wid = c * _NS + s
    for i in range(16):
        for j in range(4):
            zb[i, pl.ds(j * 16, 16)] = jnp.zeros((16,), jnp.float32)
    for j in range(40):
        pltpu.sync_copy(zb, shared.at[pl.ds(s * _SPAN + j * 16, 16), :])
    pltpu.sync_copy(src_hbm.at[pl.ds(wid * _CPW, _CPW)], isrc)
    pltpu.sync_copy(dst_hbm.at[pl.ds(wid * _CPW, _CPW)], idst)
    plsc.subcore_barrier()

    for b in range(_NBUF):
        pltpu.async_copy(hp_hbm.at[isrc.at[b]], rows.at[b], sems[b])

    def step(k, carry):
        for b in range(_NBUF):
            ch = k * _NBUF + b
            pltpu.make_async_copy(hp_hbm.at[isrc.at[b]], rows.at[b],
                                  sems[b]).wait()
            pltpu.sync_copy(rows.at[b], shared.at[idst.at[ch]], add=True)

            @pl.when(k < _CPW // _NBUF - 1)
            def _():
                pltpu.async_copy(hp_hbm.at[isrc.at[ch + _NBUF]], rows.at[b],
                                 sems[b])
        return carry

    lax.fori_loop(0, _CPW // _NBUF, step, 0)
    plsc.subcore_barrier()
    pltpu.sync_copy(shared.at[pl.ds(s * _SPAN, _SPAN), :],
                    part_hbm.at[c, pl.ds(s * _SPAN, _SPAN), :])


_spmm_call = pl.kernel(
    _spmm_body,
    out_type=jax.ShapeDtypeStruct((_NC, _NPAD, _DH), jnp.float32),
    mesh=_mesh,
    compiler_params=pltpu.CompilerParams(use_tc_tiling_on_sc=False),
    scratch_types=[
        pltpu.VMEM((_CPW, _CH), jnp.int32),
        pltpu.VMEM((_CPW, _CH), jnp.int32),
        pltpu.VMEM((_NBUF, _CH, _DH), jnp.float32),
        pltpu.VMEM((16, _DH), jnp.float32),
        pltpu.VMEM_SHARED((_NPAD, _DH), jnp.float32),
        pltpu.SemaphoreType.DMA,
        pltpu.SemaphoreType.DMA,
        pltpu.SemaphoreType.DMA,
        pltpu.SemaphoreType.DMA,
    ],
)


_BN = 2000


def _dinv_of(degt_blk):
    return lax.rsqrt(degt_blk[:, 0] + degt_blk[:, 1] + 1.0)


def _tc1_body(x_ref, w1_ref, degp_ref, hp_ref):
    dinv = _dinv_of(degp_ref[...])
    h = jnp.dot(x_ref[...], w1_ref[...], preferred_element_type=jnp.float32)
    hp_ref[...] = h * dinv[:, None]


def _tc1(x, W1, degp):
    return pl.pallas_call(
        _tc1_body,
        grid=(_N // _BN,),
        in_specs=[
            pl.BlockSpec((_BN, _DIN), lambda i: (i, 0)),
            pl.BlockSpec((_DIN, _DH), lambda i: (0, 0)),
            pl.BlockSpec((_BN, _NC), lambda i: (i, 0)),
        ],
        out_specs=pl.BlockSpec((_BN, _DH), lambda i: (i, 0)),
        out_shape=jax.ShapeDtypeStruct((_N, _DH), jnp.float32),
    )(x, W1, degp)


def _tc2_body(q_ref, hp_ref, degp_ref, b1_ref, we_ref, be_ref, wd_ref,
              bd_ref, wc_ref, bc_ref, z_ref, pred_ref, hp2_ref):
    dinv = _dinv_of(degp_ref[...])
    s1 = dinv[:, None] * (q_ref[0] + q_ref[1] + hp_ref[...]) + b1_ref[...]
    h1 = jnp.maximum(s1, 0.0)
    z = jnp.dot(h1, we_ref[...], preferred_element_type=jnp.float32) + be_ref[...]
    h2 = jnp.maximum(
        jnp.dot(z, wd_ref[...], preferred_element_type=jnp.float32) + bd_ref[...],
        0.0)
    z_ref[...] = z
    pred_ref[...] = jnp.dot(z, wc_ref[...], preferred_element_type=jnp.float32) + bc_ref[...]
    hp2_ref[...] = h2 * dinv[:, None]


def _tc2(q, hp, degp, b1, We, be, Wd, bd, Wc, bc):
    return pl.pallas_call(
        _tc2_body,
        grid=(_N // _BN,),
        in_specs=[
            pl.BlockSpec((_NC, _BN, _DH), lambda i: (0, i, 0)),
            pl.BlockSpec((_BN, _DH), lambda i: (i, 0)),
            pl.BlockSpec((_BN, _NC), lambda i: (i, 0)),
            pl.BlockSpec((1, _DH), lambda i: (0, 0)),
            pl.BlockSpec((_DH, _DL), lambda i: (0, 0)),
            pl.BlockSpec((1, _DL), lambda i: (0, 0)),
            pl.BlockSpec((_DL, _DH), lambda i: (0, 0)),
            pl.BlockSpec((1, _DH), lambda i: (0, 0)),
            pl.BlockSpec((_DL, _DOUT), lambda i: (0, 0)),
            pl.BlockSpec((1, _DOUT), lambda i: (0, 0)),
        ],
        out_specs=[
            pl.BlockSpec((_BN, _DL), lambda i: (i, 0)),
            pl.BlockSpec((_BN, _DOUT), lambda i: (i, 0)),
            pl.BlockSpec((_BN, _DH), lambda i: (i, 0)),
        ],
        out_shape=[
            jax.ShapeDtypeStruct((_N, _DL), jnp.float32),
            jax.ShapeDtypeStruct((_N, _DOUT), jnp.float32),
            jax.ShapeDtypeStruct((_N, _DH), jnp.float32),
        ],
    )(q, hp, degp, b1, We, be, Wd, bd, Wc, bc)


def _tc3_body(r_ref, hp2_ref, degp_ref, w2_ref, b2_ref, out_ref):
    dinv = _dinv_of(degp_ref[...])
    t = dinv[:, None] * (r_ref[0] + r_ref[1] + hp2_ref[...])
    out_ref[...] = jnp.dot(t, w2_ref[...], preferred_element_type=jnp.float32) + b2_ref[...]


def _tc3(r, hp2, degp, W2, b2):
    return pl.pallas_call(
        _tc3_body,
        grid=(_N // _BN,),
        in_specs=[
            pl.BlockSpec((_NC, _BN, _DH), lambda i: (0, i, 0)),
            pl.BlockSpec((_BN, _DH), lambda i: (i, 0)),
            pl.BlockSpec((_BN, _NC), lambda i: (i, 0)),
            pl.BlockSpec((_DH, _DIN), lambda i: (0, 0)),
            pl.BlockSpec((1, _DIN), lambda i: (0, 0)),
        ],
        out_specs=pl.BlockSpec((_BN, _DIN), lambda i: (i, 0)),
        out_shape=jax.ShapeDtypeStruct((_N, _DIN), jnp.float32),
    )(r, hp2, degp, W2, b2)


def kernel(x, edge_index, edge_attr, W1, b1, We, be, Wd, bd, W2, b2, Wc, bc):
    npad = _EP - _E
    iota = jnp.arange(npad, dtype=jnp.int32)
    srcp = jnp.concatenate(
        [edge_index[0], iota % _N]).reshape(_NW * _CPW, _CH)
    dstp = jnp.concatenate(
        [edge_index[1], _N + iota % (_NPAD - _N)]).reshape(_NW * _CPW, _CH)

    degp = _deg_call(dstp)
    degt = degp.T
    hp = _tc1(x, W1, degt)
    q = _spmm_call(hp, srcp, dstp)
    z, pred, hp2 = _tc2(q, hp, degt, b1.reshape(1, -1), We,
                        be.reshape(1, -1), Wd, bd.reshape(1, -1), Wc,
                        bc.reshape(1, -1))
    r = _spmm_call(hp2, srcp, dstp)
    x_recon = _tc3(r, hp2, degt, W2, b2.reshape(1, -1))
    return (x_recon, z, pred)
